# baseline (speedup 1.0000x reference)
import jax
import jax.numpy as jnp
from jax import lax
from jax.experimental import pallas as pl
from jax.experimental.pallas import tpu as pltpu
from jax.experimental.pallas import tpu_sc as plsc
import functools

N = 100000
D = 128
S = 64
NC = 2
NS = 16
CHUNK = 250
HALF = 125
NCHUNKS = N // CHUNK
SEGS_PER_C = S // NC
SEGS_PER_W = SEGS_PER_C // NS
LANES = 16
NG = D // LANES
SINK = S
SROWS = 80
NBUF = 3

_mesh = plsc.VectorSubcoreMesh(
    core_axis_name="c", subcore_axis_name="s", num_cores=NC, num_subcores=NS)


@functools.partial(
    pl.kernel,
    out_type=jax.ShapeDtypeStruct((S, D), jnp.float32),
    mesh=_mesh,
    scratch_types=[
        [pltpu.VMEM((CHUNK, D), jnp.float32) for _ in range(NBUF)],
        [pltpu.VMEM((2, HALF), jnp.int32) for _ in range(NBUF)],
        pltpu.VMEM((NCHUNKS // LANES, LANES), jnp.int32),
        pltpu.VMEM((NCHUNKS // LANES, LANES), jnp.int32),
        pltpu.VMEM((HALF, LANES), jnp.float32),
        pltpu.VMEM((S, D), jnp.float32),
        pltpu.VMEM((S, LANES), jnp.float32),
        pltpu.VMEM((1, S), jnp.int32),
        pltpu.VMEM((5, D), jnp.float32),
        pltpu.VMEM((5, LANES), jnp.float32),
        pltpu.VMEM((SEGS_PER_W, D), jnp.float32),
        pltpu.VMEM((SEGS_PER_W, LANES), jnp.float32),
        pltpu.VMEM_SHARED((SROWS, D), jnp.float32),
        pltpu.VMEM_SHARED((SROWS, LANES), jnp.float32),
        [pltpu.SemaphoreType.DMA for _ in range(NBUF)],
        [pltpu.SemaphoreType.DMA for _ in range(NBUF)],
    ],
    compiler_params=pltpu.CompilerParams(use_tc_tiling_on_sc=False),
)
def _pool_sc(x_hbm, idx_hbm, f_hbm, l_hbm, out_hbm,
             xbs, ibs, f_v, l_v, ones_v, acc_local, cnt_local,
             idbuf, zbuf, czbuf, sbuf, cbuf, shared_sum, shared_cnt,
             semxs, semis):
    cid = lax.axis_index("c")
    sid = lax.axis_index("s")

    zeros16 = jnp.zeros((LANES,), jnp.float32)
    ones16 = jnp.ones((LANES,), jnp.float32)
    izeros = jnp.zeros((LANES,), jnp.int32)
    iones = jnp.ones((LANES,), jnp.int32)

    for r in range(5):
        for g in range(NG):
            zbuf[r, pl.ds(g * LANES, LANES)] = zeros16
        czbuf[r, :] = zeros16
    pltpu.sync_copy(zbuf, shared_sum.at[pl.ds(sid * 5, 5)])
    pltpu.sync_copy(czbuf, shared_cnt.at[pl.ds(sid * 5, 5)])

    for r in range(HALF):
        ones_v[r, :] = ones16

    for g in range(S // LANES):
        idbuf[0, pl.ds(g * LANES, LANES)] = (
            lax.iota(jnp.int32, LANES) + g * LANES)

    def _zero_body(r, _):
        for g in range(NG):
            acc_local[r, pl.ds(g * LANES, LANES)] = zeros16
        cnt_local[r, :] = zeros16
        return 0
    lax.fori_loop(0, S, _zero_body, 0)

    pltpu.sync_copy(f_hbm, f_v)
    pltpu.sync_copy(l_hbm, l_v)
    cut = jnp.int32(SEGS_PER_C)

    def _count_below(v_ref):
        def _b(r, acc):
            return acc + jnp.where(v_ref[r, :] < cut, iones, izeros)
        vec = lax.fori_loop(0, NCHUNKS // LANES, _b, izeros)
        tot = vec[0]
        for i in range(1, LANES):
            tot = tot + vec[i]
        return tot

    p_cnt = _count_below(f_v)
    q_cnt = _count_below(l_v)
    lo = jnp.where(cid == 0, jnp.int32(0), q_cnt)
    hi = jnp.where(cid == 0, p_cnt, jnp.int32(NCHUNKS))
    count = hi - lo
    my_lo = lo + (count * sid) // NS
    my_hi = lo + (count * (sid + 1)) // NS
    nk = my_hi - my_lo
    seg_min = cid * cut
    seg_max = seg_min + cut

    plsc.subcore_barrier()

    def _x_copy(ck, b):
        return pltpu.make_async_copy(
            x_hbm.at[pl.ds(ck * CHUNK, CHUNK)], xbs[b], semxs[b])

    def _i_copy(ck, b):
        return pltpu.make_async_copy(
            idx_hbm.at[pl.ds(2 * ck, 2)], ibs[b], semis[b])

    for pb in range(NBUF - 1):
        @pl.when(nk > pb)
        def _prime(pb=pb):
            _x_copy(my_lo + pb, pb).start()
            _i_copy(my_lo + pb, pb).start()

    def _process(b):
        xb = xbs[b]
        ib = ibs[b]
        seg_lo = ib[0, pl.ds(0, LANES)][0]
        seg_hi = ib[1, pl.ds(HALF - LANES, LANES)][LANES - 1]
        pure = seg_lo == seg_hi

        @pl.when(pure)
        def _pure():
            def _body(r, acc):
                for u in range(5):
                    acc = tuple(
                        acc[g] + xb[r * 5 + u, pl.ds(g * LANES, LANES)]
                        for g in range(NG))
                return acc
            acc = lax.fori_loop(
                0, CHUNK // 5, _body, tuple(zeros16 for _ in range(NG)))
            for g in range(NG):
                sl = pl.ds(g * LANES, LANES)
                acc_local[seg_lo, sl] = acc_local[seg_lo, sl] + acc[g]
            cnt_local[seg_lo, :] = cnt_local[seg_lo, :] + float(CHUNK)

        @pl.when(jnp.logical_not(pure))
        def _impure():
            sink16 = jnp.full((LANES,), SINK, jnp.int32)
            for h in range(2):
                for o in list(range(0, HALF - LANES, LANES)) + [HALF - LANES]:
                    sl = pl.ds(o, LANES)
                    v = ib[h, sl]
                    ok = jnp.logical_and(v >= seg_min, v < seg_max)
                    ib[h, sl] = jnp.where(ok, v, sink16)
            for h in range(2):
                idx_row = ib.at[h]
                pltpu.sync_copy(xb.at[pl.ds(h * HALF, HALF)],
                                shared_sum.at[idx_row], add=True)
                pltpu.sync_copy(ones_v, shared_cnt.at[idx_row], add=True)

    def _loop_body(k, _):
        ck = my_lo + k

        @pl.when(k + NBUF - 1 < nk)
        def _prefetch():
            nb = (k + NBUF - 1) % NBUF
            for bb in range(NBUF):
                @pl.when(nb == bb)
                def _(bb=bb):
                    _x_copy(ck + NBUF - 1, bb).start()
                    _i_copy(ck + NBUF - 1, bb).start()

        b = k % NBUF
        for bb in range(NBUF):
            @pl.when(b == bb)
            def _(bb=bb):
                _x_copy(ck, bb).wait()
                _i_copy(ck, bb).wait()
                _process(bb)

        return 0

    lax.fori_loop(0, nk, _loop_body, 0)

    id_row = idbuf.at[0]
    pltpu.sync_copy(acc_local, shared_sum.at[id_row], add=True)
    pltpu.sync_copy(cnt_local, shared_cnt.at[id_row], add=True)

    plsc.subcore_barrier()

    seg0 = cid * SEGS_PER_C + sid * SEGS_PER_W
    pltpu.sync_copy(shared_sum.at[pl.ds(seg0, SEGS_PER_W)], sbuf)
    pltpu.sync_copy(shared_cnt.at[pl.ds(seg0, SEGS_PER_W)], cbuf)
    for r in range(SEGS_PER_W):
        cnt = jnp.maximum(cbuf[r, :], 1.0)
        for g in range(NG):
            sl = pl.ds(g * LANES, LANES)
            sbuf[r, sl] = sbuf[r, sl] / cnt
    pltpu.sync_copy(sbuf, out_hbm.at[pl.ds(seg0, SEGS_PER_W)])


def kernel(x_node_features, batch_vector):
    idx2d = batch_vector.astype(jnp.int32).reshape(NCHUNKS * 2, HALF)
    b2 = batch_vector.astype(jnp.int32).reshape(NCHUNKS, CHUNK)
    firsts = b2[:, 0].reshape(NCHUNKS // LANES, LANES)
    lasts = b2[:, CHUNK - 1].reshape(NCHUNKS // LANES, LANES)
    return _pool_sc(x_node_features, idx2d, firsts, lasts)


# speedup vs baseline: 1.1189x; 1.1189x over previous
import jax
import jax.numpy as jnp
from jax import lax
from jax.experimental import pallas as pl
from jax.experimental.pallas import tpu as pltpu
from jax.experimental.pallas import tpu_sc as plsc
import functools

N = 100000
D = 128
S = 64
NC = 2
NS = 16
CHUNK = 125
NCHUNKS = N // CHUNK
SEGS_PER_C = S // NC
SEGS_PER_W = SEGS_PER_C // NS
LANES = 16
NG = D // LANES
SINK = S
SROWS = 80
NBUF = 4

_mesh = plsc.VectorSubcoreMesh(
    core_axis_name="c", subcore_axis_name="s", num_cores=NC, num_subcores=NS)


@functools.partial(
    pl.kernel,
    out_type=jax.ShapeDtypeStruct((S, D), jnp.float32),
    mesh=_mesh,
    scratch_types=[
        [pltpu.VMEM((CHUNK, D), jnp.float32) for _ in range(NBUF)],
        [pltpu.VMEM((1, CHUNK), jnp.int32) for _ in range(NBUF)],
        pltpu.VMEM((NCHUNKS // LANES, LANES), jnp.int32),
        pltpu.VMEM((NCHUNKS // LANES, LANES), jnp.int32),
        pltpu.VMEM((CHUNK, LANES), jnp.float32),
        pltpu.VMEM((S, D), jnp.float32),
        pltpu.VMEM((S, LANES), jnp.float32),
        pltpu.VMEM((1, S), jnp.int32),
        pltpu.VMEM((5, D), jnp.float32),
        pltpu.VMEM((5, LANES), jnp.float32),
        pltpu.VMEM((SEGS_PER_W, D), jnp.float32),
        pltpu.VMEM((SEGS_PER_W, LANES), jnp.float32),
        pltpu.VMEM_SHARED((SROWS, D), jnp.float32),
        pltpu.VMEM_SHARED((SROWS, LANES), jnp.float32),
        [pltpu.SemaphoreType.DMA for _ in range(NBUF)],
        [pltpu.SemaphoreType.DMA for _ in range(NBUF)],
    ],
    compiler_params=pltpu.CompilerParams(use_tc_tiling_on_sc=False),
)
def _pool_sc(x_hbm, idx_hbm, f_hbm, l_hbm, out_hbm,
             xbs, ibs, f_v, l_v, ones_v, acc_local, cnt_local,
             idbuf, zbuf, czbuf, sbuf, cbuf, shared_sum, shared_cnt,
             semxs, semis):
    cid = lax.axis_index("c")
    sid = lax.axis_index("s")

    zeros16 = jnp.zeros((LANES,), jnp.float32)
    ones16 = jnp.ones((LANES,), jnp.float32)
    izeros = jnp.zeros((LANES,), jnp.int32)
    iones = jnp.ones((LANES,), jnp.int32)

    for r in range(5):
        for g in range(NG):
            zbuf[r, pl.ds(g * LANES, LANES)] = zeros16
        czbuf[r, :] = zeros16
    pltpu.sync_copy(zbuf, shared_sum.at[pl.ds(sid * 5, 5)])
    pltpu.sync_copy(czbuf, shared_cnt.at[pl.ds(sid * 5, 5)])

    for r in range(CHUNK):
        ones_v[r, :] = ones16

    for g in range(S // LANES):
        idbuf[0, pl.ds(g * LANES, LANES)] = (
            lax.iota(jnp.int32, LANES) + g * LANES)

    def _zero_body(r, _):
        for g in range(NG):
            acc_local[r, pl.ds(g * LANES, LANES)] = zeros16
        cnt_local[r, :] = zeros16
        return 0
    lax.fori_loop(0, S, _zero_body, 0)

    pltpu.sync_copy(f_hbm, f_v)
    pltpu.sync_copy(l_hbm, l_v)
    cut = jnp.int32(SEGS_PER_C)

    def _count_below(v_ref):
        def _b(r, acc):
            return acc + jnp.where(v_ref[r, :] < cut, iones, izeros)
        vec = lax.fori_loop(0, NCHUNKS // LANES, _b, izeros)
        tot = vec[0]
        for i in range(1, LANES):
            tot = tot + vec[i]
        return tot

    p_cnt = _count_below(f_v)
    q_cnt = _count_below(l_v)
    lo = jnp.where(cid == 0, jnp.int32(0), q_cnt)
    hi = jnp.where(cid == 0, p_cnt, jnp.int32(NCHUNKS))
    count = hi - lo
    my_lo = lo + (count * sid) // NS
    my_hi = lo + (count * (sid + 1)) // NS
    nk = my_hi - my_lo
    seg_min = cid * cut
    seg_max = seg_min + cut

    plsc.subcore_barrier()

    def _x_copy(ck, b):
        return pltpu.make_async_copy(
            x_hbm.at[pl.ds(ck * CHUNK, CHUNK)], xbs[b], semxs[b])

    def _i_copy(ck, b):
        return pltpu.make_async_copy(
            idx_hbm.at[pl.ds(ck, 1)], ibs[b], semis[b])

    for pb in range(NBUF - 1):
        @pl.when(nk > pb)
        def _prime(pb=pb):
            _x_copy(my_lo + pb, pb).start()
            _i_copy(my_lo + pb, pb).start()

    def _process(b):
        xb = xbs[b]
        ib = ibs[b]
        seg_lo = ib[0, pl.ds(0, LANES)][0]
        seg_hi = ib[0, pl.ds(CHUNK - LANES, LANES)][LANES - 1]
        pure = seg_lo == seg_hi

        @pl.when(pure)
        def _pure():
            def _body(r, acc):
                for u in range(5):
                    acc = tuple(
                        acc[g] + xb[r * 5 + u, pl.ds(g * LANES, LANES)]
                        for g in range(NG))
                return acc
            acc = lax.fori_loop(
                0, CHUNK // 5, _body, tuple(zeros16 for _ in range(NG)))
            for g in range(NG):
                sl = pl.ds(g * LANES, LANES)
                acc_local[seg_lo, sl] = acc_local[seg_lo, sl] + acc[g]
            cnt_local[seg_lo, :] = cnt_local[seg_lo, :] + float(CHUNK)

        @pl.when(jnp.logical_not(pure))
        def _impure():
            sink16 = jnp.full((LANES,), SINK, jnp.int32)
            for o in list(range(0, CHUNK - LANES, LANES)) + [CHUNK - LANES]:
                sl = pl.ds(o, LANES)
                v = ib[0, sl]
                ok = jnp.logical_and(v >= seg_min, v < seg_max)
                ib[0, sl] = jnp.where(ok, v, sink16)
            idx_row = ib.at[0]
            pltpu.sync_copy(xb, shared_sum.at[idx_row], add=True)
            pltpu.sync_copy(ones_v, shared_cnt.at[idx_row], add=True)

    def _loop_body(k, _):
        ck = my_lo + k

        @pl.when(k + NBUF - 1 < nk)
        def _prefetch():
            nb = (k + NBUF - 1) % NBUF
            for bb in range(NBUF):
                @pl.when(nb == bb)
                def _(bb=bb):
                    _x_copy(ck + NBUF - 1, bb).start()
                    _i_copy(ck + NBUF - 1, bb).start()

        b = k % NBUF
        for bb in range(NBUF):
            @pl.when(b == bb)
            def _(bb=bb):
                _x_copy(ck, bb).wait()
                _i_copy(ck, bb).wait()
                _process(bb)

        return 0

    lax.fori_loop(0, nk, _loop_body, 0)

    id_row = idbuf.at[0]
    pltpu.sync_copy(acc_local, shared_sum.at[id_row], add=True)
    pltpu.sync_copy(cnt_local, shared_cnt.at[id_row], add=True)

    plsc.subcore_barrier()

    seg0 = cid * SEGS_PER_C + sid * SEGS_PER_W
    pltpu.sync_copy(shared_sum.at[pl.ds(seg0, SEGS_PER_W)], sbuf)
    pltpu.sync_copy(shared_cnt.at[pl.ds(seg0, SEGS_PER_W)], cbuf)
    for r in range(SEGS_PER_W):
        cnt = jnp.maximum(cbuf[r, :], 1.0)
        for g in range(NG):
            sl = pl.ds(g * LANES, LANES)
            sbuf[r, sl] = sbuf[r, sl] / cnt
    pltpu.sync_copy(sbuf, out_hbm.at[pl.ds(seg0, SEGS_PER_W)])


def kernel(x_node_features, batch_vector):
    idx2d = batch_vector.astype(jnp.int32).reshape(NCHUNKS, CHUNK)
    firsts = idx2d[:, 0].reshape(NCHUNKS // LANES, LANES)
    lasts = idx2d[:, CHUNK - 1].reshape(NCHUNKS // LANES, LANES)
    return _pool_sc(x_node_features, idx2d, firsts, lasts)
